# baseline (speedup 1.0000x reference)
import numpy as np
import jax
import jax.numpy as jnp
from jax.experimental import pallas as pl
from jax.experimental.pallas import tpu as pltpu

N = 10000
F = 128
EOS = 1e-10

BR = 80
BL = 512
BT = 1024
G = pl.cdiv(N, BT)
NP = G * BT

_steps = []
for _i in range(G):
    for _j in range(_i, G):
        _steps.append((_i, _j, 0))
        if _j > _i:
            _steps.append((_i, _j, 1))
PSTEPS = len(_steps)
_SI = np.array([s[0] for s in _steps], np.int32)
_SJ = np.array([s[1] for s in _steps], np.int32)
_SP = np.array([s[2] for s in _steps], np.int32)


def _elu1(a):
    return jnp.where(a > 0, a + 1.0, jnp.exp(a))


def _lin1_kernel(x_ref, w_ref, b_ref, o_ref):
    i = pl.program_id(0)
    rows = jax.lax.broadcasted_iota(jnp.int32, (BL, 1), 0) + i * BL
    o_ref[...] = jnp.where(
        rows < N,
        jnp.dot(x_ref[...], w_ref[...], preferred_element_type=jnp.float32)
        + b_ref[...],
        0.0,
    )


def _sums_kernel(a_ref, rs_ref, cs_ref):
    i = pl.program_id(0)

    @pl.when(i == 0)
    def _():
        cs_ref[...] = jnp.zeros_like(cs_ref)

    b = _elu1(a_ref[...])
    rs_ref[...] = jnp.sum(b, axis=1, keepdims=True)
    cs_ref[...] += jnp.sum(b, axis=0, keepdims=True)


def _tile_mask(i, j):
    rows = jax.lax.broadcasted_iota(jnp.int32, (BT, BT), 0) + i * BT
    cols = jax.lax.broadcasted_iota(jnp.int32, (BT, BT), 1) + j * BT
    return (rows < N) & (cols < N)


def _main_kernel(si_ref, sj_ref, sp_ref, aij_ref, aji_ref, dcol_ref, drow_ref,
                 h1_ref, w2_ref, b2_ref, adj_ref, h2_ref, adjT_ref):
    p = pl.program_id(0)
    i = si_ref[p]
    j = sj_ref[p]
    ph = sp_ref[p]

    @pl.when(p == 0)
    def _():
        h2_ref[...] = jnp.zeros_like(h2_ref)

    @pl.when(ph == 0)
    def _():
        bij = _elu1(aij_ref[...])
        bji = _elu1(aji_ref[...])
        s = 0.5 * (bij + bji.T)
        di = dcol_ref[pl.ds(i * BT, BT), :]
        dj = drow_ref[:, pl.ds(j * BT, BT)]
        adj = jnp.where(_tile_mask(i, j), s * di * dj, 0.0)
        adj_ref[...] = adj
        adjT_ref[...] = adj.T
        h1s = h1_ref[pl.ds(j * BT, BT), :]
        contrib = jnp.dot(adj, h1s, preferred_element_type=jnp.float32)
        h2_ref[pl.ds(i * BT, BT), :] += contrib

    @pl.when(ph == 1)
    def _():
        adjT = adjT_ref[...]
        adj_ref[...] = adjT
        h1s = h1_ref[pl.ds(i * BT, BT), :]
        contrib = jnp.dot(adjT, h1s, preferred_element_type=jnp.float32)
        h2_ref[pl.ds(j * BT, BT), :] += contrib

    @pl.when(p == PSTEPS - 1)
    def _():
        h = jnp.maximum(h2_ref[...], 0.0)
        h2_ref[...] = (
            jnp.dot(h, w2_ref[...], preferred_element_type=jnp.float32)
            + b2_ref[...]
        )


def _mm_kernel(adj_ref, h2_ref, o_ref):
    o_ref[...] = jnp.dot(adj_ref[...], h2_ref[...],
                         preferred_element_type=jnp.float32)


def kernel(features, x, Adj_param, W1, b1, W2, b2):
    del features

    h1 = pl.pallas_call(
        _lin1_kernel,
        grid=(NP // BL,),
        in_specs=[
            pl.BlockSpec((BL, F), lambda i: (i, 0)),
            pl.BlockSpec((F, F), lambda i: (0, 0)),
            pl.BlockSpec((1, F), lambda i: (0, 0)),
        ],
        out_specs=pl.BlockSpec((BL, F), lambda i: (i, 0)),
        out_shape=jax.ShapeDtypeStruct((NP, F), jnp.float32),
    )(x, W1, b1.reshape(1, F))

    rs, cs = pl.pallas_call(
        _sums_kernel,
        grid=(N // BR,),
        in_specs=[pl.BlockSpec((BR, N), lambda i: (i, 0))],
        out_specs=[
            pl.BlockSpec((BR, 1), lambda i: (i, 0)),
            pl.BlockSpec((1, N), lambda i: (0, 0)),
        ],
        out_shape=[
            jax.ShapeDtypeStruct((N, 1), jnp.float32),
            jax.ShapeDtypeStruct((1, N), jnp.float32),
        ],
    )(Adj_param)

    deg = 0.5 * (rs[:, 0] + cs[0, :])
    isd = 1.0 / (jnp.sqrt(deg) + EOS)
    isd = jnp.pad(isd, (0, NP - N))
    dcol = isd[:, None]
    drow = isd[None, :]

    adj_, h2 = pl.pallas_call(
        _main_kernel,
        grid_spec=pltpu.PrefetchScalarGridSpec(
            num_scalar_prefetch=3,
            grid=(PSTEPS,),
            in_specs=[
                pl.BlockSpec((BT, BT), lambda p, si, sj, sp: (si[p], sj[p])),
                pl.BlockSpec((BT, BT), lambda p, si, sj, sp: (sj[p], si[p])),
                pl.BlockSpec((NP, 1), lambda p, si, sj, sp: (0, 0)),
                pl.BlockSpec((1, NP), lambda p, si, sj, sp: (0, 0)),
                pl.BlockSpec((NP, F), lambda p, si, sj, sp: (0, 0)),
                pl.BlockSpec((F, F), lambda p, si, sj, sp: (0, 0)),
                pl.BlockSpec((1, F), lambda p, si, sj, sp: (0, 0)),
            ],
            out_specs=[
                pl.BlockSpec(
                    (BT, BT),
                    lambda p, si, sj, sp: (
                        jnp.where(sp[p] == 0, si[p], sj[p]),
                        jnp.where(sp[p] == 0, sj[p], si[p]),
                    ),
                ),
                pl.BlockSpec((NP, F), lambda p, si, sj, sp: (0, 0)),
            ],
            scratch_shapes=[pltpu.VMEM((BT, BT), jnp.float32)],
        ),
        out_shape=[
            jax.ShapeDtypeStruct((N, N), jnp.float32),
            jax.ShapeDtypeStruct((NP, F), jnp.float32),
        ],
        compiler_params=pltpu.CompilerParams(
            dimension_semantics=("arbitrary",),
        ),
    )(jnp.asarray(_SI), jnp.asarray(_SJ), jnp.asarray(_SP),
      Adj_param, Adj_param, dcol, drow, h1, W2, b2.reshape(1, F))

    out = pl.pallas_call(
        _mm_kernel,
        grid=(N // BR,),
        in_specs=[
            pl.BlockSpec((BR, N), lambda i: (i, 0)),
            pl.BlockSpec((N, F), lambda i: (0, 0)),
        ],
        out_specs=pl.BlockSpec((BR, F), lambda i: (i, 0)),
        out_shape=jax.ShapeDtypeStruct((N, F), jnp.float32),
    )(adj_, h2[:N])

    return (out, adj_)


# speedup vs baseline: 1.1128x; 1.1128x over previous
import jax
import jax.numpy as jnp
from jax.experimental import pallas as pl
from jax.experimental.pallas import tpu as pltpu

N = 10000
F = 128
EOS = 1e-10

BR = 80
BL = 512
BI = 1024
BJ = 1024
GI = pl.cdiv(N, BI)
GJ = pl.cdiv(N, BJ)
NPI = GI * BI
NPJ = GJ * BJ


def _elu1(a):
    return jnp.where(a > 0, a + 1.0, jnp.exp(a))


def _colmask(j):
    cols = jax.lax.broadcasted_iota(jnp.int32, (1, BJ), 1) + j * BJ
    return cols < N


def _lin1_kernel(x_ref, w_ref, b_ref, o_ref):
    i = pl.program_id(0)
    rows = jax.lax.broadcasted_iota(jnp.int32, (BL, 1), 0) + i * BL
    o_ref[...] = jnp.where(
        rows < N,
        jnp.dot(x_ref[...], w_ref[...], preferred_element_type=jnp.float32)
        + b_ref[...],
        0.0,
    )


def _sums_kernel(a_ref, rs_ref, cs_ref):
    i = pl.program_id(0)

    @pl.when(i == 0)
    def _():
        cs_ref[...] = jnp.zeros_like(cs_ref)

    b = _elu1(a_ref[...])
    rs_ref[...] = jnp.sum(b, axis=1, keepdims=True)
    cs_ref[...] += jnp.sum(b, axis=0, keepdims=True)


def _main_kernel(aij_ref, aji_ref, dcol_ref, drow_ref, h1_ref, w2_ref, b2_ref,
                 adj_ref, h2_ref):
    j = pl.program_id(1)
    bij = _elu1(aij_ref[...])
    bji = _elu1(aji_ref[...])
    s = 0.5 * (bij + bji.T)
    adj = s * dcol_ref[...] * drow_ref[...]
    adj = jnp.where(_colmask(j), adj, 0.0)
    adj_ref[...] = adj
    h1s = h1_ref[pl.ds(j * BJ, BJ), :]
    contrib = jnp.dot(adj, h1s, preferred_element_type=jnp.float32)

    @pl.when(j == 0)
    def _():
        h2_ref[...] = contrib

    @pl.when(j > 0)
    def _():
        h2_ref[...] += contrib

    @pl.when(j == GJ - 1)
    def _():
        h = jnp.maximum(h2_ref[...], 0.0)
        h2_ref[...] = (
            jnp.dot(h, w2_ref[...], preferred_element_type=jnp.float32)
            + b2_ref[...]
        )


def _mm_kernel(adj_ref, h2_ref, o_ref):
    o_ref[...] = jnp.dot(adj_ref[...], h2_ref[...],
                         preferred_element_type=jnp.float32)


def kernel(features, x, Adj_param, W1, b1, W2, b2):
    del features

    h1 = pl.pallas_call(
        _lin1_kernel,
        grid=(NPJ // BL,),
        in_specs=[
            pl.BlockSpec((BL, F), lambda i: (i, 0)),
            pl.BlockSpec((F, F), lambda i: (0, 0)),
            pl.BlockSpec((1, F), lambda i: (0, 0)),
        ],
        out_specs=pl.BlockSpec((BL, F), lambda i: (i, 0)),
        out_shape=jax.ShapeDtypeStruct((NPJ, F), jnp.float32),
    )(x, W1, b1.reshape(1, F))

    rs, cs = pl.pallas_call(
        _sums_kernel,
        grid=(N // BR,),
        in_specs=[pl.BlockSpec((BR, N), lambda i: (i, 0))],
        out_specs=[
            pl.BlockSpec((BR, 1), lambda i: (i, 0)),
            pl.BlockSpec((1, N), lambda i: (0, 0)),
        ],
        out_shape=[
            jax.ShapeDtypeStruct((N, 1), jnp.float32),
            jax.ShapeDtypeStruct((1, N), jnp.float32),
        ],
    )(Adj_param)

    deg = 0.5 * (rs[:, 0] + cs[0, :])
    isd = 1.0 / (jnp.sqrt(deg) + EOS)
    dcol = isd[:, None]
    drow = isd[None, :]

    adj_, h2 = pl.pallas_call(
        _main_kernel,
        grid=(GI, GJ),
        in_specs=[
            pl.BlockSpec((BI, BJ), lambda i, j: (i, j)),
            pl.BlockSpec((BJ, BI), lambda i, j: (j, i)),
            pl.BlockSpec((BI, 1), lambda i, j: (i, 0)),
            pl.BlockSpec((1, BJ), lambda i, j: (0, j)),
            pl.BlockSpec((NPJ, F), lambda i, j: (0, 0)),
            pl.BlockSpec((F, F), lambda i, j: (0, 0)),
            pl.BlockSpec((1, F), lambda i, j: (0, 0)),
        ],
        out_specs=[
            pl.BlockSpec((BI, BJ), lambda i, j: (i, j)),
            pl.BlockSpec((BI, F), lambda i, j: (i, 0)),
        ],
        out_shape=[
            jax.ShapeDtypeStruct((N, N), jnp.float32),
            jax.ShapeDtypeStruct((N, F), jnp.float32),
        ],
    )(Adj_param, Adj_param, dcol, drow, h1, W2, b2.reshape(1, F))

    out = pl.pallas_call(
        _mm_kernel,
        grid=(N // BR,),
        in_specs=[
            pl.BlockSpec((BR, N), lambda i: (i, 0)),
            pl.BlockSpec((N, F), lambda i: (0, 0)),
        ],
        out_specs=pl.BlockSpec((BR, F), lambda i: (i, 0)),
        out_shape=jax.ShapeDtypeStruct((N, F), jnp.float32),
    )(adj_, h2)

    return (out, adj_)


# speedup vs baseline: 1.2736x; 1.1444x over previous
import jax
import jax.numpy as jnp
from jax.experimental import pallas as pl
from jax.experimental.pallas import tpu as pltpu

N = 10000
F = 128
EOS = 1e-10

BR = 200
BL = 512
BI = 1280
BJ = 1280
GI = pl.cdiv(N, BI)
GJ = pl.cdiv(N, BJ)
NPI = GI * BI
NPJ = GJ * BJ


def _elu1(a):
    return jnp.where(a > 0, a + 1.0, jnp.exp(a))


def _colmask(j):
    cols = jax.lax.broadcasted_iota(jnp.int32, (1, BJ), 1) + j * BJ
    return cols < N


def _lin1_kernel(x_ref, w_ref, b_ref, o_ref):
    i = pl.program_id(0)
    rows = jax.lax.broadcasted_iota(jnp.int32, (BL, 1), 0) + i * BL
    o_ref[...] = jnp.where(
        rows < N,
        jnp.dot(x_ref[...], w_ref[...], preferred_element_type=jnp.float32)
        + b_ref[...],
        0.0,
    )


def _sums_kernel(a_ref, rs_ref, cs_ref):
    i = pl.program_id(0)

    @pl.when(i == 0)
    def _():
        cs_ref[...] = jnp.zeros_like(cs_ref)

    b = _elu1(a_ref[...])
    rs_ref[...] = jnp.sum(b, axis=1, keepdims=True)
    cs_ref[...] += jnp.sum(b, axis=0, keepdims=True)


def _main_kernel(aij_ref, aji_ref, dcol_ref, drow_ref, h1_ref, w2_ref, b2_ref,
                 adj_ref, h2_ref):
    j = pl.program_id(1)
    bij = _elu1(aij_ref[...])
    bji = _elu1(aji_ref[...])
    s = 0.5 * (bij + bji.T)
    adj = s * dcol_ref[...] * drow_ref[...]
    adj = jnp.where(_colmask(j), adj, 0.0)
    adj_ref[...] = adj
    h1s = h1_ref[pl.ds(j * BJ, BJ), :]
    contrib = jnp.dot(adj, h1s, preferred_element_type=jnp.float32)

    @pl.when(j == 0)
    def _():
        h2_ref[...] = contrib

    @pl.when(j > 0)
    def _():
        h2_ref[...] += contrib

    @pl.when(j == GJ - 1)
    def _():
        h = jnp.maximum(h2_ref[...], 0.0)
        h2_ref[...] = (
            jnp.dot(h, w2_ref[...], preferred_element_type=jnp.float32)
            + b2_ref[...]
        )


def _mm_kernel(adj_ref, h2_ref, o_ref):
    o_ref[...] = jnp.dot(adj_ref[...], h2_ref[...],
                         preferred_element_type=jnp.float32)


def kernel(features, x, Adj_param, W1, b1, W2, b2):
    del features

    h1 = pl.pallas_call(
        _lin1_kernel,
        grid=(NPJ // BL,),
        in_specs=[
            pl.BlockSpec((BL, F), lambda i: (i, 0)),
            pl.BlockSpec((F, F), lambda i: (0, 0)),
            pl.BlockSpec((1, F), lambda i: (0, 0)),
        ],
        out_specs=pl.BlockSpec((BL, F), lambda i: (i, 0)),
        out_shape=jax.ShapeDtypeStruct((NPJ, F), jnp.float32),
    )(x, W1, b1.reshape(1, F))

    rs, cs = pl.pallas_call(
        _sums_kernel,
        grid=(N // BR,),
        in_specs=[pl.BlockSpec((BR, N), lambda i: (i, 0))],
        out_specs=[
            pl.BlockSpec((BR, 1), lambda i: (i, 0)),
            pl.BlockSpec((1, N), lambda i: (0, 0)),
        ],
        out_shape=[
            jax.ShapeDtypeStruct((N, 1), jnp.float32),
            jax.ShapeDtypeStruct((1, N), jnp.float32),
        ],
    )(Adj_param)

    deg = 0.5 * (rs[:, 0] + cs[0, :])
    isd = 1.0 / (jnp.sqrt(deg) + EOS)
    dcol = isd[:, None]
    drow = isd[None, :]

    adj_, h2 = pl.pallas_call(
        _main_kernel,
        grid=(GI, GJ),
        in_specs=[
            pl.BlockSpec((BI, BJ), lambda i, j: (i, j)),
            pl.BlockSpec((BJ, BI), lambda i, j: (j, i)),
            pl.BlockSpec((BI, 1), lambda i, j: (i, 0)),
            pl.BlockSpec((1, BJ), lambda i, j: (0, j)),
            pl.BlockSpec((NPJ, F), lambda i, j: (0, 0)),
            pl.BlockSpec((F, F), lambda i, j: (0, 0)),
            pl.BlockSpec((1, F), lambda i, j: (0, 0)),
        ],
        out_specs=[
            pl.BlockSpec((BI, BJ), lambda i, j: (i, j)),
            pl.BlockSpec((BI, F), lambda i, j: (i, 0)),
        ],
        out_shape=[
            jax.ShapeDtypeStruct((N, N), jnp.float32),
            jax.ShapeDtypeStruct((N, F), jnp.float32),
        ],
    )(Adj_param, Adj_param, dcol, drow, h1, W2, b2.reshape(1, F))

    out = pl.pallas_call(
        _mm_kernel,
        grid=(N // BR,),
        in_specs=[
            pl.BlockSpec((BR, N), lambda i: (i, 0)),
            pl.BlockSpec((N, F), lambda i: (0, 0)),
        ],
        out_specs=pl.BlockSpec((BR, F), lambda i: (i, 0)),
        out_shape=jax.ShapeDtypeStruct((N, F), jnp.float32),
    )(adj_, h2)

    return (out, adj_)


# speedup vs baseline: 1.2959x; 1.0175x over previous
import jax
import jax.numpy as jnp
from jax.experimental import pallas as pl
from jax.experimental.pallas import tpu as pltpu

N = 10000
F = 128
EOS = 1e-10

BR = 400
BL = 512
BI = 1280
BJ = 1280
GI = pl.cdiv(N, BI)
GJ = pl.cdiv(N, BJ)
NPI = GI * BI
NPJ = GJ * BJ


def _elu1(a):
    return jnp.where(a > 0, a + 1.0, jnp.exp(a))


def _colmask(j):
    cols = jax.lax.broadcasted_iota(jnp.int32, (1, BJ), 1) + j * BJ
    return cols < N


def _lin1_kernel(x_ref, w_ref, b_ref, o_ref):
    i = pl.program_id(0)
    rows = jax.lax.broadcasted_iota(jnp.int32, (BL, 1), 0) + i * BL
    o_ref[...] = jnp.where(
        rows < N,
        jnp.dot(x_ref[...], w_ref[...], preferred_element_type=jnp.float32)
        + b_ref[...],
        0.0,
    )


def _sums_kernel(a_ref, rs_ref, cs_ref):
    i = pl.program_id(0)

    @pl.when(i == 0)
    def _():
        cs_ref[...] = jnp.zeros_like(cs_ref)

    b = _elu1(a_ref[...])
    rs_ref[...] = jnp.sum(b, axis=1, keepdims=True)
    cs_ref[...] += jnp.sum(b, axis=0, keepdims=True)


def _main_kernel(aij_ref, aji_ref, dcol_ref, drow_ref, h1_ref, w2_ref, b2_ref,
                 adj_ref, h2_ref):
    j = pl.program_id(1)
    bij = _elu1(aij_ref[...])
    bji = _elu1(aji_ref[...])
    s = 0.5 * (bij + bji.T)
    adj = s * dcol_ref[...] * drow_ref[...]
    adj = jnp.where(_colmask(j), adj, 0.0)
    adj_ref[...] = adj
    h1s = h1_ref[pl.ds(j * BJ, BJ), :]
    contrib = jnp.dot(adj, h1s, preferred_element_type=jnp.float32)

    @pl.when(j == 0)
    def _():
        h2_ref[...] = contrib

    @pl.when(j > 0)
    def _():
        h2_ref[...] += contrib

    @pl.when(j == GJ - 1)
    def _():
        h = jnp.maximum(h2_ref[...], 0.0)
        h2_ref[...] = (
            jnp.dot(h, w2_ref[...], preferred_element_type=jnp.float32)
            + b2_ref[...]
        )


def _mm_kernel(adj_ref, h2_ref, o_ref):
    o_ref[...] = jnp.dot(adj_ref[...], h2_ref[...],
                         preferred_element_type=jnp.float32)


def kernel(features, x, Adj_param, W1, b1, W2, b2):
    del features

    h1 = pl.pallas_call(
        _lin1_kernel,
        grid=(NPJ // BL,),
        in_specs=[
            pl.BlockSpec((BL, F), lambda i: (i, 0)),
            pl.BlockSpec((F, F), lambda i: (0, 0)),
            pl.BlockSpec((1, F), lambda i: (0, 0)),
        ],
        out_specs=pl.BlockSpec((BL, F), lambda i: (i, 0)),
        out_shape=jax.ShapeDtypeStruct((NPJ, F), jnp.float32),
    )(x, W1, b1.reshape(1, F))

    rs, cs = pl.pallas_call(
        _sums_kernel,
        grid=(N // BR,),
        in_specs=[pl.BlockSpec((BR, N), lambda i: (i, 0))],
        out_specs=[
            pl.BlockSpec((BR, 1), lambda i: (i, 0)),
            pl.BlockSpec((1, N), lambda i: (0, 0)),
        ],
        out_shape=[
            jax.ShapeDtypeStruct((N, 1), jnp.float32),
            jax.ShapeDtypeStruct((1, N), jnp.float32),
        ],
    )(Adj_param)

    deg = 0.5 * (rs[:, 0] + cs[0, :])
    isd = 1.0 / (jnp.sqrt(deg) + EOS)
    dcol = isd[:, None]
    drow = isd[None, :]

    adj_, h2 = pl.pallas_call(
        _main_kernel,
        grid=(GI, GJ),
        in_specs=[
            pl.BlockSpec((BI, BJ), lambda i, j: (i, j)),
            pl.BlockSpec((BJ, BI), lambda i, j: (j, i)),
            pl.BlockSpec((BI, 1), lambda i, j: (i, 0)),
            pl.BlockSpec((1, BJ), lambda i, j: (0, j)),
            pl.BlockSpec((NPJ, F), lambda i, j: (0, 0)),
            pl.BlockSpec((F, F), lambda i, j: (0, 0)),
            pl.BlockSpec((1, F), lambda i, j: (0, 0)),
        ],
        out_specs=[
            pl.BlockSpec((BI, BJ), lambda i, j: (i, j)),
            pl.BlockSpec((BI, F), lambda i, j: (i, 0)),
        ],
        out_shape=[
            jax.ShapeDtypeStruct((N, N), jnp.float32),
            jax.ShapeDtypeStruct((N, F), jnp.float32),
        ],
    )(Adj_param, Adj_param, dcol, drow, h1, W2, b2.reshape(1, F))

    out = pl.pallas_call(
        _mm_kernel,
        grid=(N // BR,),
        in_specs=[
            pl.BlockSpec((BR, N), lambda i: (i, 0)),
            pl.BlockSpec((N, F), lambda i: (0, 0)),
        ],
        out_specs=pl.BlockSpec((BR, F), lambda i: (i, 0)),
        out_shape=jax.ShapeDtypeStruct((N, F), jnp.float32),
    )(adj_, h2)

    return (out, adj_)


# speedup vs baseline: 1.3291x; 1.0257x over previous
import numpy as np
import jax
import jax.numpy as jnp
from jax.experimental import pallas as pl
from jax.experimental.pallas import tpu as pltpu

N = 10000
F = 128
EOS = 1e-10

BR = 400
BL = 512
BT = 1024
G = pl.cdiv(N, BT)
NP = G * BT

_steps = []
for _i in range(G):
    for _j in range(_i, G):
        if _i == _j:
            _steps.append((_i, _j, 2))
        else:
            _steps.append((_i, _j, 0))
            _steps.append((_i, _j, 1))
PSTEPS = len(_steps)

_aii = [0] * PSTEPS
_aij = [0] * PSTEPS
_aji = [0] * PSTEPS
_ajj = [0] * PSTEPS
for _s, (_i, _j, _t) in enumerate(_steps):
    if _t in (0, 2):
        _aii[_s], _aij[_s] = _i, _j
        _aji[_s], _ajj[_s] = _j, _i
for _s, (_i, _j, _t) in enumerate(_steps):
    if _t == 1:
        if _s + 1 < PSTEPS:
            _aii[_s], _aij[_s] = _aii[_s + 1], _aij[_s + 1]
        else:
            _aii[_s], _aij[_s] = _i, _j
        _aji[_s], _ajj[_s] = _j, _i
for _s, (_i, _j, _t) in enumerate(_steps):
    if _t == 2:
        _nxt = None
        for _s2 in range(_s + 1, PSTEPS):
            if _steps[_s2][2] == 0:
                _nxt = _s2
                break
        if _nxt is not None:
            _aji[_s], _ajj[_s] = _aji[_nxt], _ajj[_nxt]
        elif _s > 0:
            _aji[_s], _ajj[_s] = _aji[_s - 1], _ajj[_s - 1]

_SI = np.array([s[0] for s in _steps], np.int32)
_SJ = np.array([s[1] for s in _steps], np.int32)
_TP = np.array([s[2] for s in _steps], np.int32)
_AII = np.array(_aii, np.int32)
_AIJ = np.array(_aij, np.int32)
_AJI = np.array(_aji, np.int32)
_AJJ = np.array(_ajj, np.int32)
_OI = np.array([(i if t != 1 else j) for (i, j, t) in _steps], np.int32)
_OJ = np.array([(j if t != 1 else i) for (i, j, t) in _steps], np.int32)


def _elu1(a):
    return jnp.where(a > 0, a + 1.0, jnp.exp(a))


def _tile_mask(i, j):
    rows = jax.lax.broadcasted_iota(jnp.int32, (BT, BT), 0) + i * BT
    cols = jax.lax.broadcasted_iota(jnp.int32, (BT, BT), 1) + j * BT
    return (rows < N) & (cols < N)


def _lin1_kernel(x_ref, w_ref, b_ref, o_ref):
    i = pl.program_id(0)
    rows = jax.lax.broadcasted_iota(jnp.int32, (BL, 1), 0) + i * BL
    o_ref[...] = jnp.where(
        rows < N,
        jnp.dot(x_ref[...], w_ref[...], preferred_element_type=jnp.float32)
        + b_ref[...],
        0.0,
    )


def _sums_kernel(a_ref, rs_ref, cs_ref):
    i = pl.program_id(0)

    @pl.when(i == 0)
    def _():
        cs_ref[...] = jnp.zeros_like(cs_ref)

    b = _elu1(a_ref[...])
    rs_ref[...] = jnp.sum(b, axis=1, keepdims=True)
    cs_ref[...] += jnp.sum(b, axis=0, keepdims=True)


def _main_kernel(aii_r, aij_r, aji_r, ajj_r, oi_r, oj_r, si_r, sj_r, tp_r,
                 a_ij_ref, a_ji_ref, dcol_ref, drow_ref, h1_ref, w2_ref,
                 b2_ref, adj_ref, h2_ref, adjT_ref):
    p = pl.program_id(0)
    i = si_r[p]
    j = sj_r[p]
    t = tp_r[p]

    @pl.when(p == 0)
    def _():
        h2_ref[...] = jnp.zeros_like(h2_ref)

    @pl.when(t == 0)
    def _():
        bij = _elu1(a_ij_ref[...])
        bji = _elu1(a_ji_ref[...])
        s = 0.5 * (bij + bji.T)
        di = dcol_ref[pl.ds(i * BT, BT), :]
        dj = drow_ref[:, pl.ds(j * BT, BT)]
        adj = jnp.where(_tile_mask(i, j), s * di * dj, 0.0)
        adj_ref[...] = adj
        adjT_ref[...] = adj.T
        h1s = h1_ref[pl.ds(j * BT, BT), :]
        contrib = jnp.dot(adj, h1s, preferred_element_type=jnp.float32)
        h2_ref[pl.ds(i * BT, BT), :] += contrib

    @pl.when(t == 1)
    def _():
        adjT = adjT_ref[...]
        adj_ref[...] = adjT
        h1s = h1_ref[pl.ds(i * BT, BT), :]
        contrib = jnp.dot(adjT, h1s, preferred_element_type=jnp.float32)
        h2_ref[pl.ds(j * BT, BT), :] += contrib

    @pl.when(t == 2)
    def _():
        bii = _elu1(a_ij_ref[...])
        s = 0.5 * (bii + bii.T)
        di = dcol_ref[pl.ds(i * BT, BT), :]
        dj = drow_ref[:, pl.ds(i * BT, BT)]
        adj = jnp.where(_tile_mask(i, i), s * di * dj, 0.0)
        adj_ref[...] = adj
        h1s = h1_ref[pl.ds(i * BT, BT), :]
        contrib = jnp.dot(adj, h1s, preferred_element_type=jnp.float32)
        h2_ref[pl.ds(i * BT, BT), :] += contrib

    @pl.when(p == PSTEPS - 1)
    def _():
        h = jnp.maximum(h2_ref[...], 0.0)
        h2_ref[...] = (
            jnp.dot(h, w2_ref[...], preferred_element_type=jnp.float32)
            + b2_ref[...]
        )


def _mm_kernel(adj_ref, h2_ref, o_ref):
    o_ref[...] = jnp.dot(adj_ref[...], h2_ref[...],
                         preferred_element_type=jnp.float32)


def kernel(features, x, Adj_param, W1, b1, W2, b2):
    del features

    h1 = pl.pallas_call(
        _lin1_kernel,
        grid=(NP // BL,),
        in_specs=[
            pl.BlockSpec((BL, F), lambda i: (i, 0)),
            pl.BlockSpec((F, F), lambda i: (0, 0)),
            pl.BlockSpec((1, F), lambda i: (0, 0)),
        ],
        out_specs=pl.BlockSpec((BL, F), lambda i: (i, 0)),
        out_shape=jax.ShapeDtypeStruct((NP, F), jnp.float32),
    )(x, W1, b1.reshape(1, F))

    rs, cs = pl.pallas_call(
        _sums_kernel,
        grid=(N // BR,),
        in_specs=[pl.BlockSpec((BR, N), lambda i: (i, 0))],
        out_specs=[
            pl.BlockSpec((BR, 1), lambda i: (i, 0)),
            pl.BlockSpec((1, N), lambda i: (0, 0)),
        ],
        out_shape=[
            jax.ShapeDtypeStruct((N, 1), jnp.float32),
            jax.ShapeDtypeStruct((1, N), jnp.float32),
        ],
    )(Adj_param)

    deg = 0.5 * (rs[:, 0] + cs[0, :])
    isd = 1.0 / (jnp.sqrt(deg) + EOS)
    isd = jnp.pad(isd, (0, NP - N))
    dcol = isd[:, None]
    drow = isd[None, :]

    adj_, h2 = pl.pallas_call(
        _main_kernel,
        grid_spec=pltpu.PrefetchScalarGridSpec(
            num_scalar_prefetch=9,
            grid=(PSTEPS,),
            in_specs=[
                pl.BlockSpec(
                    (BT, BT),
                    lambda p, aii, aij, aji, ajj, oi, oj, si, sj, tp:
                        (aii[p], aij[p]),
                ),
                pl.BlockSpec(
                    (BT, BT),
                    lambda p, aii, aij, aji, ajj, oi, oj, si, sj, tp:
                        (aji[p], ajj[p]),
                ),
                pl.BlockSpec(
                    (NP, 1),
                    lambda p, aii, aij, aji, ajj, oi, oj, si, sj, tp: (0, 0),
                ),
                pl.BlockSpec(
                    (1, NP),
                    lambda p, aii, aij, aji, ajj, oi, oj, si, sj, tp: (0, 0),
                ),
                pl.BlockSpec(
                    (NP, F),
                    lambda p, aii, aij, aji, ajj, oi, oj, si, sj, tp: (0, 0),
                ),
                pl.BlockSpec(
                    (F, F),
                    lambda p, aii, aij, aji, ajj, oi, oj, si, sj, tp: (0, 0),
                ),
                pl.BlockSpec(
                    (1, F),
                    lambda p, aii, aij, aji, ajj, oi, oj, si, sj, tp: (0, 0),
                ),
            ],
            out_specs=[
                pl.BlockSpec(
                    (BT, BT),
                    lambda p, aii, aij, aji, ajj, oi, oj, si, sj, tp:
                        (oi[p], oj[p]),
                ),
                pl.BlockSpec(
                    (NP, F),
                    lambda p, aii, aij, aji, ajj, oi, oj, si, sj, tp: (0, 0),
                ),
            ],
            scratch_shapes=[pltpu.VMEM((BT, BT), jnp.float32)],
        ),
        out_shape=[
            jax.ShapeDtypeStruct((N, N), jnp.float32),
            jax.ShapeDtypeStruct((NP, F), jnp.float32),
        ],
        compiler_params=pltpu.CompilerParams(
            dimension_semantics=("arbitrary",),
        ),
    )(jnp.asarray(_AII), jnp.asarray(_AIJ), jnp.asarray(_AJI),
      jnp.asarray(_AJJ), jnp.asarray(_OI), jnp.asarray(_OJ),
      jnp.asarray(_SI), jnp.asarray(_SJ), jnp.asarray(_TP),
      Adj_param, Adj_param, dcol, drow, h1, W2, b2.reshape(1, F))

    out = pl.pallas_call(
        _mm_kernel,
        grid=(N // BR,),
        in_specs=[
            pl.BlockSpec((BR, N), lambda i: (i, 0)),
            pl.BlockSpec((N, F), lambda i: (0, 0)),
        ],
        out_specs=pl.BlockSpec((BR, F), lambda i: (i, 0)),
        out_shape=jax.ShapeDtypeStruct((N, F), jnp.float32),
    )(adj_, h2[:N])

    return (out, adj_)
